# baseline (speedup 1.0000x reference)
import jax
import jax.numpy as jnp
from jax import lax
from jax.experimental import pallas as pl
from jax.experimental.pallas import tpu as pltpu
from jax.experimental.pallas import tpu_sc as plsc

D_DIM = 32
NI = 4096
NJ = 200
NUM_CORES = 2
NUM_SUBCORES = 16
JG = 8
IG = 4
JPW = NJ // JG
IPW = NI // IG
CH = 512
HALVES = IPW // CH
ITEMS = JPW * HALVES
KT = D_DIM // 8
ITL = CH // 128
SROW = 129
ITLP = ITL + 1
TPR = 4


NVT = 1000000 // 128
VTAIL = 1000000 - NVT * 128
TCW = [NVT // 32 + (1 if w < NVT % 32 else 0) for w in range(32)]


def _fmt_body(wt_hbm, tail_hbm, out_hbm, g_v, s_v, s3_v, t_v, t2_v,
              gsem, ssem):
    c = lax.axis_index("c")
    s = lax.axis_index("s")
    w = s * NUM_CORES + c
    nw = NVT // 32
    ex = NVT % 32
    cnt = nw + jnp.where(w < ex, 1, 0)
    base = w * nw + jnp.minimum(w, ex)

    iota = lax.iota(jnp.int32, 16)
    vvs = [iota + 16 * gz for gz in range(8)]

    def load_start(tc, b):
        pltpu.async_copy(wt_hbm.at[:, pl.ds(tc * 128, 128)], g_v.at[b],
                         gsem.at[b])

    def load_wait(tc, b):
        pltpu.make_async_copy(wt_hbm.at[:, pl.ds(tc * 128, 128)], g_v.at[b],
                              gsem.at[b]).wait()

    def store_start(tc, b):
        pltpu.async_copy(s3_v.at[b],
                         out_hbm.at[pl.ds(tc * 128 * D_DIM, 128 * D_DIM)],
                         ssem.at[b])

    def store_wait(tc, b):
        pltpu.make_async_copy(
            s3_v.at[b],
            out_hbm.at[pl.ds(tc * 128 * D_DIM, 128 * D_DIM)],
            ssem.at[b]).wait()

    def transpose(b):
        gv = g_v.at[b]
        sv = s_v.at[b]

        def kbody(k, _):
            kv = iota - iota + k
            for gz in range(8):
                plsc.store_scatter(sv, [vvs[gz], kv], gv[k, pl.ds(16 * gz, 16)])
            return 0

        lax.fori_loop(0, D_DIM, kbody, 0)

        def vbody(v4, _):
            for u in range(4):
                v = v4 * 4 + u
                s3_v[b, pl.ds(v * D_DIM, 16)] = sv[v, pl.ds(0, 16)]
                s3_v[b, pl.ds(v * D_DIM + 16, 16)] = sv[v, pl.ds(16, 16)]
            return 0

        lax.fori_loop(0, 32, vbody, 0)

    @pl.when(cnt >= 1)
    def _():
        load_start(base, 0)

    @pl.when(cnt >= 2)
    def _():
        load_start(base + 1, 1)

    def outer(t, _):
        for par in (0, 1):
            tc = base + 2 * t + par
            bb = par

            @pl.when(2 * t + par < cnt)
            def _(tc=tc, bb=bb):
                load_wait(tc, bb)

                @pl.when(2 * t + par >= 2)
                def _(tc=tc, bb=bb):
                    store_wait(tc - 2, bb)

                transpose(bb)
                store_start(tc, bb)

                @pl.when(2 * t + par + 2 < cnt)
                def _(tc=tc, bb=bb):
                    load_start(tc + 2, bb)
        return 0

    lax.fori_loop(0, (nw + 2) // 2, outer, 0)

    @pl.when(cnt >= 2)
    def _():
        store_wait(base + cnt - 2, (cnt - 2) % 2)

    @pl.when(cnt >= 1)
    def _():
        store_wait(base + cnt - 1, (cnt - 1) % 2)

    @pl.when(w == 0)
    def _():
        pltpu.sync_copy(tail_hbm, t_v)

        def tbody(r, _):
            t2_v[pl.ds(r * D_DIM, 16)] = t_v[r, pl.ds(0, 16)]
            t2_v[pl.ds(r * D_DIM + 16, 16)] = t_v[r, pl.ds(16, 16)]
            return 0

        lax.fori_loop(0, VTAIL, tbody, 0)
        pltpu.sync_copy(t2_v,
                        out_hbm.at[pl.ds(NVT * 128 * D_DIM, VTAIL * D_DIM)])


def _emb_body(w_hbm, idx_hbm, out_hbm, idx_v, g_v, s_v, gsem, ssem):
    c = lax.axis_index("c")
    s = lax.axis_index("s")
    w = s * NUM_CORES + c
    jg = w // IG
    ig = w % IG
    j0 = jg * JPW
    it_base = ig * (IPW // 128)

    pltpu.sync_copy(idx_hbm.at[pl.ds(j0, JPW), pl.ds(ig * IPW, IPW)], idx_v)

    iota = lax.iota(jnp.int32, 16)
    ktv0 = iota // 8
    krv0 = iota % 8
    ktv1 = (iota + 16) // 8
    krv1 = (iota + 16) % 8
    zerov = iota - iota
    rows0 = [ktv0 * (ITLP * 8) + itl * 8 + krv0 for itl in range(ITL)]
    rows1 = [ktv1 * (ITLP * 8) + itl * 8 + krv1 for itl in range(ITL)]

    def start_item(g, b):
        jl = g // HALVES
        half = g % HALVES
        pltpu.async_copy(w_hbm.at[idx_v.at[jl, pl.ds(half * CH, CH)]],
                         g_v.at[b], gsem.at[b])

    def wait_item(g, b):
        jl = g // HALVES
        half = g % HALVES
        pltpu.make_async_copy(w_hbm.at[idx_v.at[jl, pl.ds(half * CH, CH)]],
                              g_v.at[b], gsem.at[b]).wait()

    def transpose(b):
        gv = g_v.at[b]
        sv = s_v.at[b]
        for itl in range(ITL):
            r0v = rows0[itl]
            r1v = rows1[itl]

            def ilbody(z, _, r0v=r0v, r1v=r1v, itl=itl):
                for u in range(8):
                    il = z * 8 + u
                    r = itl * 128 + il
                    ilv = zerov + il
                    plsc.store_scatter(sv, [r0v, ilv],
                                       gv[r, pl.ds(0, 16)])
                    plsc.store_scatter(sv, [r1v, ilv],
                                       gv[r, pl.ds(16, 16)])
                return 0

            lax.fori_loop(0, 16, ilbody, 0)

    def store_start(g, b):
        jl = g // HALVES
        half = g % HALVES
        for kt in range(KT):
            for itl in range(ITL):
                pltpu.async_copy(
                    s_v.at[b, pl.ds((kt * ITLP + itl) * 8, 8), pl.ds(0, 128)],
                    out_hbm.at[j0 + jl, kt, it_base + half * ITL + itl],
                    ssem.at[b])

    def store_wait(g, b):
        jl = g // HALVES
        half = g % HALVES
        for kt in range(KT):
            for itl in range(ITL):
                pltpu.make_async_copy(
                    s_v.at[b, pl.ds((kt * ITLP + itl) * 8, 8), pl.ds(0, 128)],
                    out_hbm.at[j0 + jl, kt, it_base + half * ITL + itl],
                    ssem.at[b]).wait()

    start_item(0, 0)
    start_item(1, 1)

    def outer(t, _):
        for par, b in ((0, 0), (1, 1)):
            g = 2 * t + par
            wait_item(g, b)

            @pl.when(t >= 1)
            def _(g=g, b=b):
                store_wait(g - 2, b)

            transpose(b)
            store_start(g, b)

            @pl.when(t <= ITEMS // 2 - 2)
            def _(g=g, b=b):
                start_item(g + 2, b)
        return 0

    lax.fori_loop(0, ITEMS // 2, outer, 0)
    store_wait(ITEMS - 2, 0)
    store_wait(ITEMS - 1, 1)


def kernel(weight, token_ids):
    idx_t = token_ids.T.astype(jnp.int32)
    mesh = plsc.VectorSubcoreMesh(core_axis_name="c", subcore_axis_name="s")
    tail = jnp.pad(weight[NVT * 128:], ((0, 0), (0, 128 - D_DIM)))
    w_lin = pl.kernel(
        _fmt_body,
        out_type=jax.ShapeDtypeStruct(((NVT * 128 + VTAIL) * D_DIM,),
                                      jnp.float32),
        mesh=mesh,
        scratch_types=[
            pltpu.VMEM((2, D_DIM, 128), jnp.float32),
            pltpu.VMEM((2, 128, 33), jnp.float32),
            pltpu.VMEM((2, 128 * D_DIM), jnp.float32),
            pltpu.VMEM((VTAIL, 128), jnp.float32),
            pltpu.VMEM((VTAIL * D_DIM,), jnp.float32),
            pltpu.SemaphoreType.DMA((2,)),
            pltpu.SemaphoreType.DMA((2,)),
        ],
        compiler_params=pltpu.CompilerParams(use_tc_tiling_on_sc=True,
                                             needs_layout_passes=False),
    )(weight.T, tail)
    w_table = w_lin.reshape(NVT * 128 + VTAIL, D_DIM)
    out_p = pl.kernel(
        _emb_body,
        out_type=jax.ShapeDtypeStruct((NJ, KT, NI // 128, 8, 128),
                                      jnp.float32),
        mesh=mesh,
        scratch_types=[
            pltpu.VMEM((JPW, IPW), jnp.int32),
            pltpu.VMEM((2, CH, D_DIM), jnp.float32),
            pltpu.VMEM((2, KT * ITLP * 8, SROW), jnp.float32),
            pltpu.SemaphoreType.DMA((2,)),
            pltpu.SemaphoreType.DMA((2,)),
        ],
        compiler_params=pltpu.CompilerParams(use_tc_tiling_on_sc=False,
                                             needs_layout_passes=False),
    )(w_table, idx_t)
    return out_p.transpose(2, 4, 0, 1, 3).reshape(NI, NJ, D_DIM)


# speedup vs baseline: 1.0006x; 1.0006x over previous
import jax
import jax.numpy as jnp
from jax import lax
from jax.experimental import pallas as pl
from jax.experimental.pallas import tpu as pltpu
from jax.experimental.pallas import tpu_sc as plsc

D_DIM = 32
NI = 4096
NJ = 200
NUM_CORES = 2
NUM_SUBCORES = 16
JG = 8
IG = 4
JPW = NJ // JG
IPW = NI // IG
CH = 512
HALVES = IPW // CH
ITEMS = JPW * HALVES
KT = D_DIM // 8
ITL = CH // 128
SROW = 129
ITLP = ITL + 1
TPR = 4


NVT = 1000000 // 128
VTAIL = 1000000 - NVT * 128
TCW = [NVT // 32 + (1 if w < NVT % 32 else 0) for w in range(32)]


def _fmt_body(wt_hbm, tail_hbm, out_hbm, g_v, s_v, s3_v, t_v, t2_v,
              gsem, ssem):
    c = lax.axis_index("c")
    s = lax.axis_index("s")
    w = s * NUM_CORES + c
    nw = NVT // 32
    ex = NVT % 32
    cnt = nw + jnp.where(w < ex, 1, 0)
    base = w * nw + jnp.minimum(w, ex)

    iota = lax.iota(jnp.int32, 16)
    vvs = [iota + 16 * gz for gz in range(8)]

    def load_start(tc, b):
        pltpu.async_copy(wt_hbm.at[:, pl.ds(tc * 128, 128)], g_v.at[b],
                         gsem.at[b])

    def load_wait(tc, b):
        pltpu.make_async_copy(wt_hbm.at[:, pl.ds(tc * 128, 128)], g_v.at[b],
                              gsem.at[b]).wait()

    def store_start(tc, b):
        pltpu.async_copy(s3_v.at[b],
                         out_hbm.at[pl.ds(tc * 128 * D_DIM, 128 * D_DIM)],
                         ssem.at[b])

    def store_wait(tc, b):
        pltpu.make_async_copy(
            s3_v.at[b],
            out_hbm.at[pl.ds(tc * 128 * D_DIM, 128 * D_DIM)],
            ssem.at[b]).wait()

    def transpose(b):
        gv = g_v.at[b]
        sv = s_v.at[b]

        for k in range(D_DIM):
            kv = iota - iota + k
            for gz in range(8):
                plsc.store_scatter(sv, [vvs[gz], kv], gv[k, pl.ds(16 * gz, 16)])

        def vbody(v8, _):
            for u in range(8):
                v = v8 * 8 + u
                s3_v[b, pl.ds(v * D_DIM, 16)] = sv[v, pl.ds(0, 16)]
                s3_v[b, pl.ds(v * D_DIM + 16, 16)] = sv[v, pl.ds(16, 16)]
            return 0

        lax.fori_loop(0, 16, vbody, 0)

    @pl.when(cnt >= 1)
    def _():
        load_start(base, 0)

    @pl.when(cnt >= 2)
    def _():
        load_start(base + 1, 1)

    def outer(t, _):
        for par in (0, 1):
            tc = base + 2 * t + par
            bb = par

            @pl.when(2 * t + par < cnt)
            def _(tc=tc, bb=bb):
                load_wait(tc, bb)

                @pl.when(2 * t + par >= 2)
                def _(tc=tc, bb=bb):
                    store_wait(tc - 2, bb)

                transpose(bb)
                store_start(tc, bb)

                @pl.when(2 * t + par + 2 < cnt)
                def _(tc=tc, bb=bb):
                    load_start(tc + 2, bb)
        return 0

    lax.fori_loop(0, (nw + 2) // 2, outer, 0)

    @pl.when(cnt >= 2)
    def _():
        store_wait(base + cnt - 2, (cnt - 2) % 2)

    @pl.when(cnt >= 1)
    def _():
        store_wait(base + cnt - 1, (cnt - 1) % 2)

    @pl.when(w == 0)
    def _():
        pltpu.sync_copy(tail_hbm, t_v)

        def tbody(r, _):
            t2_v[pl.ds(r * D_DIM, 16)] = t_v[r, pl.ds(0, 16)]
            t2_v[pl.ds(r * D_DIM + 16, 16)] = t_v[r, pl.ds(16, 16)]
            return 0

        lax.fori_loop(0, VTAIL, tbody, 0)
        pltpu.sync_copy(t2_v,
                        out_hbm.at[pl.ds(NVT * 128 * D_DIM, VTAIL * D_DIM)])


def _emb_body(w_hbm, idx_hbm, out_hbm, idx_v, g_v, s_v, gsem, ssem):
    c = lax.axis_index("c")
    s = lax.axis_index("s")
    w = s * NUM_CORES + c
    jg = w // IG
    ig = w % IG
    j0 = jg * JPW
    it_base = ig * (IPW // 128)

    pltpu.sync_copy(idx_hbm.at[pl.ds(j0, JPW), pl.ds(ig * IPW, IPW)], idx_v)

    iota = lax.iota(jnp.int32, 16)
    ktv0 = iota // 8
    krv0 = iota % 8
    ktv1 = (iota + 16) // 8
    krv1 = (iota + 16) % 8
    zerov = iota - iota
    rows0 = [ktv0 * (ITLP * 8) + itl * 8 + krv0 for itl in range(ITL)]
    rows1 = [ktv1 * (ITLP * 8) + itl * 8 + krv1 for itl in range(ITL)]

    def start_item(g, b):
        jl = g // HALVES
        half = g % HALVES
        pltpu.async_copy(w_hbm.at[idx_v.at[jl, pl.ds(half * CH, CH)]],
                         g_v.at[b], gsem.at[b])

    def wait_item(g, b):
        jl = g // HALVES
        half = g % HALVES
        pltpu.make_async_copy(w_hbm.at[idx_v.at[jl, pl.ds(half * CH, CH)]],
                              g_v.at[b], gsem.at[b]).wait()

    def transpose(b):
        gv = g_v.at[b]
        sv = s_v.at[b]
        for itl in range(ITL):
            r0v = rows0[itl]
            r1v = rows1[itl]

            def ilbody(z, _, r0v=r0v, r1v=r1v, itl=itl):
                for u in range(8):
                    il = z * 8 + u
                    r = itl * 128 + il
                    ilv = zerov + il
                    plsc.store_scatter(sv, [r0v, ilv],
                                       gv[r, pl.ds(0, 16)])
                    plsc.store_scatter(sv, [r1v, ilv],
                                       gv[r, pl.ds(16, 16)])
                return 0

            lax.fori_loop(0, 16, ilbody, 0)

    def store_start(g, b):
        jl = g // HALVES
        half = g % HALVES
        for kt in range(KT):
            for itl in range(ITL):
                pltpu.async_copy(
                    s_v.at[b, pl.ds((kt * ITLP + itl) * 8, 8), pl.ds(0, 128)],
                    out_hbm.at[j0 + jl, kt, it_base + half * ITL + itl],
                    ssem.at[b])

    def store_wait(g, b):
        jl = g // HALVES
        half = g % HALVES
        for kt in range(KT):
            for itl in range(ITL):
                pltpu.make_async_copy(
                    s_v.at[b, pl.ds((kt * ITLP + itl) * 8, 8), pl.ds(0, 128)],
                    out_hbm.at[j0 + jl, kt, it_base + half * ITL + itl],
                    ssem.at[b]).wait()

    start_item(0, 0)
    start_item(1, 1)

    def outer(t, _):
        for par, b in ((0, 0), (1, 1)):
            g = 2 * t + par
            wait_item(g, b)

            @pl.when(t >= 1)
            def _(g=g, b=b):
                store_wait(g - 2, b)

            transpose(b)
            store_start(g, b)

            @pl.when(t <= ITEMS // 2 - 2)
            def _(g=g, b=b):
                start_item(g + 2, b)
        return 0

    lax.fori_loop(0, ITEMS // 2, outer, 0)
    store_wait(ITEMS - 2, 0)
    store_wait(ITEMS - 1, 1)


def kernel(weight, token_ids):
    idx_t = token_ids.T.astype(jnp.int32)
    mesh = plsc.VectorSubcoreMesh(core_axis_name="c", subcore_axis_name="s")
    tail = jnp.pad(weight[NVT * 128:], ((0, 0), (0, 128 - D_DIM)))
    w_lin = pl.kernel(
        _fmt_body,
        out_type=jax.ShapeDtypeStruct(((NVT * 128 + VTAIL) * D_DIM,),
                                      jnp.float32),
        mesh=mesh,
        scratch_types=[
            pltpu.VMEM((2, D_DIM, 128), jnp.float32),
            pltpu.VMEM((2, 128, 33), jnp.float32),
            pltpu.VMEM((2, 128 * D_DIM), jnp.float32),
            pltpu.VMEM((VTAIL, 128), jnp.float32),
            pltpu.VMEM((VTAIL * D_DIM,), jnp.float32),
            pltpu.SemaphoreType.DMA((2,)),
            pltpu.SemaphoreType.DMA((2,)),
        ],
        compiler_params=pltpu.CompilerParams(use_tc_tiling_on_sc=True,
                                             needs_layout_passes=False),
    )(weight.T, tail)
    w_table = w_lin.reshape(NVT * 128 + VTAIL, D_DIM)
    out_p = pl.kernel(
        _emb_body,
        out_type=jax.ShapeDtypeStruct((NJ, KT, NI // 128, 8, 128),
                                      jnp.float32),
        mesh=mesh,
        scratch_types=[
            pltpu.VMEM((JPW, IPW), jnp.int32),
            pltpu.VMEM((2, CH, D_DIM), jnp.float32),
            pltpu.VMEM((2, KT * ITLP * 8, SROW), jnp.float32),
            pltpu.SemaphoreType.DMA((2,)),
            pltpu.SemaphoreType.DMA((2,)),
        ],
        compiler_params=pltpu.CompilerParams(use_tc_tiling_on_sc=False,
                                             needs_layout_passes=False),
    )(w_table, idx_t)
    return out_p.transpose(2, 4, 0, 1, 3).reshape(NI, NJ, D_DIM)


# speedup vs baseline: 1.5502x; 1.5493x over previous
import jax
import jax.numpy as jnp
from jax import lax
from jax.experimental import pallas as pl
from jax.experimental.pallas import tpu as pltpu
from jax.experimental.pallas import tpu_sc as plsc

D_DIM = 32
NI = 4096
NJ = 200
NUM_CORES = 2
NUM_SUBCORES = 16
JG = 8
IG = 4
JPW = NJ // JG
IPW = NI // IG
CH = 512
HALVES = IPW // CH
ITEMS = JPW * HALVES
KT = D_DIM // 8
ITL = CH // 128
SROW = 129
ITLP = ITL + 1
TPR = 4


def _emb_body(w_hbm, idx_hbm, out_hbm, idx_v, g_v, s_v, gsem, ssem):
    c = lax.axis_index("c")
    s = lax.axis_index("s")
    w = s * NUM_CORES + c
    jg = w // IG
    ig = w % IG
    j0 = jg * JPW
    it_base = ig * (IPW // 128)

    pltpu.sync_copy(idx_hbm.at[pl.ds(j0, JPW), pl.ds(ig * IPW, IPW)], idx_v)

    iota = lax.iota(jnp.int32, 16)
    ktv0 = iota // 8
    krv0 = iota % 8
    ktv1 = (iota + 16) // 8
    krv1 = (iota + 16) % 8
    zerov = iota - iota
    rows0 = [ktv0 * (ITLP * 8) + itl * 8 + krv0 for itl in range(ITL)]
    rows1 = [ktv1 * (ITLP * 8) + itl * 8 + krv1 for itl in range(ITL)]

    def start_item(g, b):
        jl = g // HALVES
        half = g % HALVES
        pltpu.async_copy(w_hbm.at[idx_v.at[jl, pl.ds(half * CH, CH)]],
                         g_v.at[b], gsem.at[b])

    def wait_item(g, b):
        jl = g // HALVES
        half = g % HALVES
        pltpu.make_async_copy(w_hbm.at[idx_v.at[jl, pl.ds(half * CH, CH)]],
                              g_v.at[b], gsem.at[b]).wait()

    def transpose(b):
        gv = g_v.at[b]
        sv = s_v.at[b]
        for itl in range(ITL):
            r0v = rows0[itl]
            r1v = rows1[itl]

            def ilbody(z, _, r0v=r0v, r1v=r1v, itl=itl):
                for u in range(8):
                    il = z * 8 + u
                    r = itl * 128 + il
                    ilv = zerov + il
                    plsc.store_scatter(sv, [r0v, ilv],
                                       gv[r, pl.ds(0, 16)])
                    plsc.store_scatter(sv, [r1v, ilv],
                                       gv[r, pl.ds(16, 16)])
                return 0

            lax.fori_loop(0, 16, ilbody, 0)

    def store_start(g, b):
        jl = g // HALVES
        half = g % HALVES
        for kt in range(KT):
            for itl in range(ITL):
                pltpu.async_copy(
                    s_v.at[b, pl.ds((kt * ITLP + itl) * 8, 8), pl.ds(0, 128)],
                    out_hbm.at[j0 + jl, kt, it_base + half * ITL + itl],
                    ssem.at[b])

    def store_wait(g, b):
        jl = g // HALVES
        half = g % HALVES
        for kt in range(KT):
            for itl in range(ITL):
                pltpu.make_async_copy(
                    s_v.at[b, pl.ds((kt * ITLP + itl) * 8, 8), pl.ds(0, 128)],
                    out_hbm.at[j0 + jl, kt, it_base + half * ITL + itl],
                    ssem.at[b]).wait()

    start_item(0, 0)
    start_item(1, 1)

    def outer(t, _):
        for par, b in ((0, 0), (1, 1)):
            g = 2 * t + par
            wait_item(g, b)

            @pl.when(t >= 1)
            def _(g=g, b=b):
                store_wait(g - 2, b)

            transpose(b)
            store_start(g, b)

            @pl.when(t <= ITEMS // 2 - 2)
            def _(g=g, b=b):
                start_item(g + 2, b)
        return 0

    lax.fori_loop(0, ITEMS // 2, outer, 0)
    store_wait(ITEMS - 2, 0)
    store_wait(ITEMS - 1, 1)


def kernel(weight, token_ids):
    idx_t = token_ids.T.astype(jnp.int32)
    mesh = plsc.VectorSubcoreMesh(core_axis_name="c", subcore_axis_name="s")
    out_p = pl.kernel(
        _emb_body,
        out_type=jax.ShapeDtypeStruct((NJ, KT, NI // 128, 8, 128),
                                      jnp.float32),
        mesh=mesh,
        scratch_types=[
            pltpu.VMEM((JPW, IPW), jnp.int32),
            pltpu.VMEM((2, CH, D_DIM), jnp.float32),
            pltpu.VMEM((2, KT * ITLP * 8, SROW), jnp.float32),
            pltpu.SemaphoreType.DMA((2,)),
            pltpu.SemaphoreType.DMA((2,)),
        ],
        compiler_params=pltpu.CompilerParams(use_tc_tiling_on_sc=False,
                                             needs_layout_passes=False),
    )(weight, idx_t)
    return out_p.transpose(2, 4, 0, 1, 3).reshape(NI, NJ, D_DIM)
